# baseline (speedup 1.0000x reference)
import functools

import jax
import jax.numpy as jnp
from jax import lax
from jax.experimental import pallas as pl
from jax.experimental.pallas import tpu as pltpu
from jax.experimental.pallas import tpu_sc as plsc

_SEQ = 4096
_B = 4
_D = 1024
_NC = 2
_NS = 16
_NW = _NC * _NS

_S_CH = 8
_SEQ_W = _SEQ // _NW
_N_CH = _SEQ_W // _S_CH
_XCH = _S_CH * _B * _D
_PCH = _S_CH * _D
_NBUF = 3


def _compute_chunk(xr, pr):
    def body(n, _):
        s = n // (_D // 16)
        k = n % (_D // 16)
        po = s * _D + k * 16
        pv = pr[pl.ds(po, 16)]
        for b in range(_B):
            xo = s * _B * _D + b * _D + k * 16
            xr[pl.ds(xo, 16)] = xr[pl.ds(xo, 16)] + pv
        return 0

    lax.fori_loop(0, _S_CH * (_D // 16), body, 0)


def _sc_body(x_hbm, pe_hbm, out_hbm, *scratch):
    xbufs = scratch[0:_NBUF]
    pbufs = scratch[_NBUF:2 * _NBUF]
    xsems = scratch[2 * _NBUF:3 * _NBUF]
    psems = scratch[3 * _NBUF:4 * _NBUF]
    osems = scratch[4 * _NBUF:5 * _NBUF]

    wid = lax.axis_index("s") * _NC + lax.axis_index("c")
    xbase = wid * _SEQ_W * _B * _D
    pbase = wid * _SEQ_W * _D

    def load(i):
        j = i % _NBUF
        xl = pltpu.async_copy(
            x_hbm.at[pl.ds(xbase + i * _XCH, _XCH)], xbufs[j], xsems[j])
        plc = pltpu.async_copy(
            pe_hbm.at[pl.ds(pbase + i * _PCH, _PCH)], pbufs[j], psems[j])
        return xl, plc

    loads = {}
    stores = {}
    loads[0] = load(0)
    for i in range(_N_CH):
        j = i % _NBUF
        if i + 1 < _N_CH:
            jn = (i + 1) % _NBUF
            if (i + 1) >= _NBUF:
                stores[i + 1 - _NBUF].wait()
            loads[i + 1] = load(i + 1)
        xl, plc = loads.pop(i)
        xl.wait()
        plc.wait()
        _compute_chunk(xbufs[j], pbufs[j])
        stores[i] = pltpu.async_copy(
            xbufs[j], out_hbm.at[pl.ds(xbase + i * _XCH, _XCH)], osems[j])
    for i in range(_N_CH - _NBUF, _N_CH):
        if i >= 0:
            stores[i].wait()


def kernel(x, pe):
    seq_len, batch, d_model = x.shape
    x1 = x.reshape(-1)
    pe1 = pe.reshape(-1)
    mesh = plsc.VectorSubcoreMesh(core_axis_name="c", subcore_axis_name="s")
    scratch = (
        [pltpu.VMEM((_XCH,), jnp.float32) for _ in range(_NBUF)]
        + [pltpu.VMEM((_PCH,), jnp.float32) for _ in range(_NBUF)]
        + [pltpu.SemaphoreType.DMA for _ in range(3 * _NBUF)]
    )
    f = pl.kernel(
        _sc_body,
        out_type=jax.ShapeDtypeStruct((seq_len * batch * d_model,), x.dtype),
        mesh=mesh,
        scratch_types=scratch,
    )
    out = f(x1, pe1)
    return out.reshape(seq_len, batch, d_model)


# speedup vs baseline: 4.6996x; 4.6996x over previous
import jax
import jax.numpy as jnp
from jax.experimental import pallas as pl
from jax.experimental.pallas import tpu as pltpu

_SEQ_BLOCK = 256


def _add_pe_block(x_ref, pe_ref, o_ref):
    o_ref[...] = x_ref[...] + pe_ref[...][:, None, :]


def kernel(x, pe):
    seq_len, batch, d_model = x.shape
    grid = (seq_len // _SEQ_BLOCK,)
    return pl.pallas_call(
        _add_pe_block,
        grid=grid,
        in_specs=[
            pl.BlockSpec((_SEQ_BLOCK, batch, d_model), lambda g: (g, 0, 0)),
            pl.BlockSpec((_SEQ_BLOCK, d_model), lambda g: (g, 0)),
        ],
        out_specs=pl.BlockSpec((_SEQ_BLOCK, batch, d_model), lambda g: (g, 0, 0)),
        out_shape=jax.ShapeDtypeStruct((seq_len, batch, d_model), x.dtype),
    )(x, pe)


# speedup vs baseline: 4.7533x; 1.0114x over previous
import jax
import jax.numpy as jnp
from jax.experimental import pallas as pl
from jax.experimental.pallas import tpu as pltpu

_SEQ_BLOCK = 512


def _add_pe_block(x_ref, pe_ref, o_ref):
    o_ref[...] = x_ref[...] + pe_ref[...][:, None, :]


def kernel(x, pe):
    seq_len, batch, d_model = x.shape
    grid = (seq_len // _SEQ_BLOCK,)
    return pl.pallas_call(
        _add_pe_block,
        grid=grid,
        in_specs=[
            pl.BlockSpec((_SEQ_BLOCK, batch, d_model), lambda g: (g, 0, 0)),
            pl.BlockSpec((_SEQ_BLOCK, d_model), lambda g: (g, 0)),
        ],
        out_specs=pl.BlockSpec((_SEQ_BLOCK, batch, d_model), lambda g: (g, 0, 0)),
        out_shape=jax.ShapeDtypeStruct((seq_len, batch, d_model), x.dtype),
    )(x, pe)
